# baseline (speedup 1.0000x reference)
import numpy as np
import jax
import jax.numpy as jnp
from jax.experimental import pallas as pl
from jax.experimental.pallas import tpu as pltpu

N = 16384
IN = 4
NF = 6
FEAT = 256
PH = 128
E = 8
EH = 256
OUT = 1

TILE = 512
OMEGA = 30.0


def _moe_kernel(x_ref, fr_ref, w1_ref, eb1_ref, ew2_ref, eb2_ref,
                pw1_ref, pb1_ref, pw2_ref, pb2_ref, pw3_ref, pb3_ref,
                pwh_ref, pbh_ref,
                w0f_ref, w0p_ref, b0_ref, w1e_ref, b1e_ref,
                w2e_ref, b2e_ref, wo_ref, bo_ref,
                out_ref):
    x = x_ref[...]
    f32 = jnp.float32

    xr = jnp.repeat(x, NF, axis=1) * fr_ref[...]
    enc = jnp.concatenate([x, jnp.sin(xr), jnp.cos(xr)], axis=1)

    h = jnp.sin(OMEGA * (jnp.dot(enc, w1_ref[...],
                                 preferred_element_type=f32) + eb1_ref[...]))
    feat = jnp.sin(OMEGA * (jnp.dot(h, ew2_ref[...],
                                    preferred_element_type=f32) + eb2_ref[...]))

    p = jnp.sin(OMEGA * (jnp.dot(x, pw1_ref[...],
                                 preferred_element_type=f32) + pb1_ref[...]))
    p = jnp.sin(OMEGA * (jnp.dot(p, pw2_ref[...],
                                 preferred_element_type=f32) + pb2_ref[...]))
    p = jnp.sin(OMEGA * (jnp.dot(p, pw3_ref[...],
                                 preferred_element_type=f32) + pb3_ref[...]))
    logits = jnp.dot(p, pwh_ref[...], preferred_element_type=f32) + pbh_ref[...]

    iota = jax.lax.broadcasted_iota(jnp.int32, logits.shape, 1)
    m1 = jnp.max(logits, axis=-1, keepdims=True)
    idx1 = jnp.min(jnp.where(logits >= m1, iota, E), axis=-1, keepdims=True)
    mask1 = iota == idx1
    rest = jnp.where(mask1, -jnp.inf, logits)
    m2 = jnp.max(rest, axis=-1, keepdims=True)
    idx2 = jnp.min(jnp.where(rest >= m2, iota, E), axis=-1, keepdims=True)
    mask2 = iota == idx2
    e2 = jnp.exp(m2 - m1)
    g1 = 1.0 / (1.0 + e2)
    g2 = e2 / (1.0 + e2)
    w = jnp.where(mask1, g1, 0.0) + jnp.where(mask2, g2, 0.0)

    acc = jnp.zeros((x.shape[0], OUT), f32)
    for e in range(E):
        h0 = jnp.sin(OMEGA * (
            jnp.dot(feat, w0f_ref[e], preferred_element_type=f32)
            + jnp.dot(p, w0p_ref[e], preferred_element_type=f32)
            + b0_ref[e]))
        h1 = jnp.sin(OMEGA * (jnp.dot(h0, w1e_ref[e],
                                      preferred_element_type=f32) + b1e_ref[e]))
        h2 = jnp.sin(OMEGA * (jnp.dot(h1, w2e_ref[e],
                                      preferred_element_type=f32) + b2e_ref[e]))
        y = jnp.dot(h2, wo_ref[e], preferred_element_type=f32) + bo_ref[e]
        acc = acc + w[:, e:e + 1] * y
    out_ref[...] = acc


def kernel(x, enc_W1, enc_b1, enc_W2, enc_b2,
           pol_W1, pol_b1, pol_W2, pol_b2, pol_W3, pol_b3, pol_Wh, pol_bh,
           exp_W0, exp_b0, exp_W1, exp_b1, exp_W2, exp_b2, exp_Wo, exp_bo):
    f32 = jnp.float32
    freqs = (2.0 ** np.arange(NF, dtype=np.float32)) * np.pi
    fr = np.tile(freqs, IN).reshape(1, IN * NF)
    fr = jnp.asarray(fr)

    w0f = exp_W0[:, :FEAT, :]
    w0p = exp_W0[:, FEAT:, :]

    def row2(a):
        return a.reshape(1, -1).astype(f32)

    full = lambda shape: pl.BlockSpec(shape, lambda i: (0,) * len(shape))

    grid = (N // TILE,)
    out = pl.pallas_call(
        _moe_kernel,
        grid=grid,
        in_specs=[
            pl.BlockSpec((TILE, IN), lambda i: (i, 0)),
            full(fr.shape), full(enc_W1.shape),
            full((1, FEAT)), full(enc_W2.shape), full((1, FEAT)),
            full(pol_W1.shape), full((1, PH)), full(pol_W2.shape),
            full((1, PH)), full(pol_W3.shape), full((1, PH)),
            full(pol_Wh.shape), full((1, E)),
            full(w0f.shape), full(w0p.shape), full(exp_b0.shape),
            full(exp_W1.shape), full(exp_b1.shape),
            full(exp_W2.shape), full(exp_b2.shape),
            full(exp_Wo.shape), full(exp_bo.shape),
        ],
        out_specs=pl.BlockSpec((TILE, OUT), lambda i: (i, 0)),
        out_shape=jax.ShapeDtypeStruct((N, OUT), f32),
        compiler_params=pltpu.CompilerParams(
            dimension_semantics=("arbitrary",),
        ),
    )(x, fr, enc_W1, row2(enc_b1), enc_W2, row2(enc_b2),
      pol_W1, row2(pol_b1), pol_W2, row2(pol_b2), pol_W3, row2(pol_b3),
      pol_Wh, row2(pol_bh),
      w0f, w0p, exp_b0, exp_W1, exp_b1, exp_W2, exp_b2, exp_Wo, exp_bo)
    return out


# speedup vs baseline: 2.5680x; 2.5680x over previous
import numpy as np
import jax
import jax.numpy as jnp
from jax.experimental import pallas as pl
from jax.experimental.pallas import tpu as pltpu

N = 16384
IN = 4
NF = 6
FEAT = 256
PH = 128
E = 8
EH = 256
OUT = 1

TILE = 512
OMEGA = 30.0

_PIO2 = np.pi / 2
_C1 = np.float32(np.floor(_PIO2 * 2 ** 11) / 2 ** 11)
_C2 = np.float32(np.floor((_PIO2 - float(_C1)) * 2 ** 28) / 2 ** 28)
_C3 = np.float32(_PIO2 - float(_C1) - float(_C2))
_TWO_OVER_PI = np.float32(2.0 / np.pi)
_MAGIC = np.float32(1.5 * 2 ** 23)
_MAGIC_I = np.int32(0x4B400000)
_S1 = np.float32(-1.66666546e-1)
_S2 = np.float32(8.33216087e-3)
_S3 = np.float32(-1.95152959e-4)
_K1 = np.float32(-0.5)
_K2 = np.float32(4.16666418e-2)
_K3 = np.float32(-1.38873162e-3)
_K4 = np.float32(2.44331571e-5)


def _fast_sin(x):
    qf = x * _TWO_OVER_PI + _MAGIC
    qi = jax.lax.bitcast_convert_type(qf, jnp.int32) - _MAGIC_I
    q = qi.astype(jnp.float32)
    r = x - q * _C1
    r = r - q * _C2
    r = r - q * _C3
    r2 = r * r
    sp = r + r * (r2 * (_S1 + r2 * (_S2 + r2 * _S3)))
    cp = 1.0 + r2 * (_K1 + r2 * (_K2 + r2 * (_K3 + r2 * _K4)))
    res = jnp.where((qi & 1) == 1, cp, sp)
    return jnp.where((qi & 2) == 2, -res, res)


def _moe_kernel(x_ref, fr_ref, w1_ref, eb1_ref, ew2_ref, eb2_ref,
                pw1_ref, pb1_ref, pw2_ref, pb2_ref, pw3_ref, pb3_ref,
                pwh_ref, pbh_ref,
                w0f_ref, w0p_ref, b0_ref, w1e_ref, b1e_ref,
                w2e_ref, b2e_ref, wo_ref, bo_ref,
                out_ref):
    x = x_ref[...]
    f32 = jnp.float32

    xr = jnp.repeat(x, NF, axis=1) * fr_ref[...]
    enc = jnp.concatenate([x, jnp.sin(xr), jnp.cos(xr)], axis=1)

    h = jnp.sin(OMEGA * (jnp.dot(enc, w1_ref[...],
                                 preferred_element_type=f32) + eb1_ref[...]))
    feat = jnp.sin(OMEGA * (jnp.dot(h, ew2_ref[...],
                                    preferred_element_type=f32) + eb2_ref[...]))

    p = jnp.sin(OMEGA * (jnp.dot(x, pw1_ref[...],
                                 preferred_element_type=f32) + pb1_ref[...]))
    p = jnp.sin(OMEGA * (jnp.dot(p, pw2_ref[...],
                                 preferred_element_type=f32) + pb2_ref[...]))
    p = jnp.sin(OMEGA * (jnp.dot(p, pw3_ref[...],
                                 preferred_element_type=f32) + pb3_ref[...]))
    logits = jnp.dot(p, pwh_ref[...], preferred_element_type=f32) + pbh_ref[...]

    iota = jax.lax.broadcasted_iota(jnp.int32, logits.shape, 1)
    m1 = jnp.max(logits, axis=-1, keepdims=True)
    idx1 = jnp.min(jnp.where(logits >= m1, iota, E), axis=-1, keepdims=True)
    mask1 = iota == idx1
    rest = jnp.where(mask1, -jnp.inf, logits)
    m2 = jnp.max(rest, axis=-1, keepdims=True)
    idx2 = jnp.min(jnp.where(rest >= m2, iota, E), axis=-1, keepdims=True)
    mask2 = iota == idx2
    e2 = jnp.exp(m2 - m1)
    g1 = 1.0 / (1.0 + e2)
    g2 = e2 / (1.0 + e2)
    w = jnp.where(mask1, g1, 0.0) + jnp.where(mask2, g2, 0.0)

    acc = jnp.zeros((x.shape[0], OUT), f32)
    for e in range(E):
        h0 = _fast_sin(OMEGA * (
            jnp.dot(feat, w0f_ref[e], preferred_element_type=f32)
            + jnp.dot(p, w0p_ref[e], preferred_element_type=f32)
            + b0_ref[e]))
        h1 = _fast_sin(OMEGA * (jnp.dot(h0, w1e_ref[e],
                                        preferred_element_type=f32) + b1e_ref[e]))
        h2 = _fast_sin(OMEGA * (jnp.dot(h1, w2e_ref[e],
                                        preferred_element_type=f32) + b2e_ref[e]))
        y = jnp.dot(h2, wo_ref[e], preferred_element_type=f32) + bo_ref[e]
        acc = acc + w[:, e:e + 1] * y
    out_ref[...] = acc


def kernel(x, enc_W1, enc_b1, enc_W2, enc_b2,
           pol_W1, pol_b1, pol_W2, pol_b2, pol_W3, pol_b3, pol_Wh, pol_bh,
           exp_W0, exp_b0, exp_W1, exp_b1, exp_W2, exp_b2, exp_Wo, exp_bo):
    f32 = jnp.float32
    freqs = (2.0 ** np.arange(NF, dtype=np.float32)) * np.pi
    fr = np.tile(freqs, IN).reshape(1, IN * NF)
    fr = jnp.asarray(fr)

    w0f = exp_W0[:, :FEAT, :]
    w0p = exp_W0[:, FEAT:, :]

    def row2(a):
        return a.reshape(1, -1).astype(f32)

    full = lambda shape: pl.BlockSpec(shape, lambda i: (0,) * len(shape))

    grid = (N // TILE,)
    out = pl.pallas_call(
        _moe_kernel,
        grid=grid,
        in_specs=[
            pl.BlockSpec((TILE, IN), lambda i: (i, 0)),
            full(fr.shape), full(enc_W1.shape),
            full((1, FEAT)), full(enc_W2.shape), full((1, FEAT)),
            full(pol_W1.shape), full((1, PH)), full(pol_W2.shape),
            full((1, PH)), full(pol_W3.shape), full((1, PH)),
            full(pol_Wh.shape), full((1, E)),
            full(w0f.shape), full(w0p.shape), full(exp_b0.shape),
            full(exp_W1.shape), full(exp_b1.shape),
            full(exp_W2.shape), full(exp_b2.shape),
            full(exp_Wo.shape), full(exp_bo.shape),
        ],
        out_specs=pl.BlockSpec((TILE, OUT), lambda i: (i, 0)),
        out_shape=jax.ShapeDtypeStruct((N, OUT), f32),
        compiler_params=pltpu.CompilerParams(
            dimension_semantics=("arbitrary",),
        ),
    )(x, fr, enc_W1, row2(enc_b1), enc_W2, row2(enc_b2),
      pol_W1, row2(pol_b1), pol_W2, row2(pol_b2), pol_W3, row2(pol_b3),
      pol_Wh, row2(pol_bh),
      w0f, w0p, exp_b0, exp_W1, exp_b1, exp_W2, exp_b2, exp_Wo, exp_bo)
    return out


# speedup vs baseline: 2.7586x; 1.0742x over previous
import numpy as np
import jax
import jax.numpy as jnp
from jax.experimental import pallas as pl
from jax.experimental.pallas import tpu as pltpu

N = 16384
IN = 4
NF = 6
FEAT = 256
PH = 128
E = 8
EH = 256
OUT = 1

TILE = 512
OMEGA = 30.0

_PIO2 = np.pi / 2
_C1 = np.float32(np.floor(_PIO2 * 2 ** 11) / 2 ** 11)
_C2 = np.float32(np.floor((_PIO2 - float(_C1)) * 2 ** 28) / 2 ** 28)
_C3 = np.float32(_PIO2 - float(_C1) - float(_C2))
_TWO_OVER_PI = np.float32(2.0 / np.pi)
_MAGIC = np.float32(1.5 * 2 ** 23)
_MAGIC_I = np.int32(0x4B400000)
_S1 = np.float32(-1.66666546e-1)
_S2 = np.float32(8.33216087e-3)
_S3 = np.float32(-1.95152959e-4)
_K1 = np.float32(-0.5)
_K2 = np.float32(4.16666418e-2)
_K3 = np.float32(-1.38873162e-3)
_K4 = np.float32(2.44331571e-5)


def _fast_sin(x):
    qf = x * _TWO_OVER_PI + _MAGIC
    qi = jax.lax.bitcast_convert_type(qf, jnp.int32) - _MAGIC_I
    q = qi.astype(jnp.float32)
    r = x - q * _C1
    r = r - q * _C2
    r = r - q * _C3
    r2 = r * r
    sp = r + r * (r2 * (_S1 + r2 * (_S2 + r2 * _S3)))
    cp = 1.0 + r2 * (_K1 + r2 * (_K2 + r2 * (_K3 + r2 * _K4)))
    res = jnp.where((qi & 1) == 1, cp, sp)
    return jnp.where((qi & 2) == 2, -res, res)


def _moe_kernel(x_ref, fr_ref, w1_ref, eb1_ref, ew2_ref, eb2_ref,
                pw1_ref, pb1_ref, pw2_ref, pb2_ref, pw3_ref, pb3_ref,
                pwh_ref, pbh_ref,
                w0f_ref, w0p_ref, b0_ref, w1e_ref, b1e_ref,
                w2e_ref, b2e_ref, wo_ref, bo_ref,
                out_ref):
    x = x_ref[...]
    f32 = jnp.float32

    xr = jnp.repeat(x, NF, axis=1) * fr_ref[...]
    enc = jnp.concatenate([x, jnp.sin(xr), jnp.cos(xr)], axis=1)

    h = jnp.sin(OMEGA * (jnp.dot(enc, w1_ref[...],
                                 preferred_element_type=f32) + eb1_ref[...]))
    feat = _fast_sin(OMEGA * (jnp.dot(h, ew2_ref[...],
                                      preferred_element_type=f32) + eb2_ref[...]))

    p = jnp.sin(OMEGA * (jnp.dot(x, pw1_ref[...],
                                 preferred_element_type=f32) + pb1_ref[...]))
    p = jnp.sin(OMEGA * (jnp.dot(p, pw2_ref[...],
                                 preferred_element_type=f32) + pb2_ref[...]))
    p = _fast_sin(OMEGA * (jnp.dot(p, pw3_ref[...],
                                   preferred_element_type=f32) + pb3_ref[...]))
    logits = jnp.dot(p, pwh_ref[...], preferred_element_type=f32) + pbh_ref[...]

    iota = jax.lax.broadcasted_iota(jnp.int32, logits.shape, 1)
    m1 = jnp.max(logits, axis=-1, keepdims=True)
    idx1 = jnp.min(jnp.where(logits >= m1, iota, E), axis=-1, keepdims=True)
    mask1 = iota == idx1
    rest = jnp.where(mask1, -jnp.inf, logits)
    m2 = jnp.max(rest, axis=-1, keepdims=True)
    idx2 = jnp.min(jnp.where(rest >= m2, iota, E), axis=-1, keepdims=True)
    mask2 = iota == idx2
    e2 = jnp.exp(m2 - m1)
    g1 = 1.0 / (1.0 + e2)
    g2 = e2 / (1.0 + e2)
    w = jnp.where(mask1, g1, 0.0) + jnp.where(mask2, g2, 0.0)

    acc = jnp.zeros((x.shape[0], OUT), f32)
    for e in range(E):
        h0 = _fast_sin(OMEGA * (
            jnp.dot(feat, w0f_ref[e], preferred_element_type=f32)
            + jnp.dot(p, w0p_ref[e], preferred_element_type=f32)
            + b0_ref[e]))
        h1 = _fast_sin(OMEGA * (jnp.dot(h0, w1e_ref[e],
                                        preferred_element_type=f32) + b1e_ref[e]))
        h2 = _fast_sin(OMEGA * (jnp.dot(h1, w2e_ref[e],
                                        preferred_element_type=f32) + b2e_ref[e]))
        y = jnp.dot(h2, wo_ref[e], preferred_element_type=f32) + bo_ref[e]
        acc = acc + w[:, e:e + 1] * y
    out_ref[...] = acc


def kernel(x, enc_W1, enc_b1, enc_W2, enc_b2,
           pol_W1, pol_b1, pol_W2, pol_b2, pol_W3, pol_b3, pol_Wh, pol_bh,
           exp_W0, exp_b0, exp_W1, exp_b1, exp_W2, exp_b2, exp_Wo, exp_bo):
    f32 = jnp.float32
    freqs = (2.0 ** np.arange(NF, dtype=np.float32)) * np.pi
    fr = np.tile(freqs, IN).reshape(1, IN * NF)
    fr = jnp.asarray(fr)

    w0f = exp_W0[:, :FEAT, :]
    w0p = exp_W0[:, FEAT:, :]

    def row2(a):
        return a.reshape(1, -1).astype(f32)

    full = lambda shape: pl.BlockSpec(shape, lambda i: (0,) * len(shape))

    grid = (N // TILE,)
    out = pl.pallas_call(
        _moe_kernel,
        grid=grid,
        in_specs=[
            pl.BlockSpec((TILE, IN), lambda i: (i, 0)),
            full(fr.shape), full(enc_W1.shape),
            full((1, FEAT)), full(enc_W2.shape), full((1, FEAT)),
            full(pol_W1.shape), full((1, PH)), full(pol_W2.shape),
            full((1, PH)), full(pol_W3.shape), full((1, PH)),
            full(pol_Wh.shape), full((1, E)),
            full(w0f.shape), full(w0p.shape), full(exp_b0.shape),
            full(exp_W1.shape), full(exp_b1.shape),
            full(exp_W2.shape), full(exp_b2.shape),
            full(exp_Wo.shape), full(exp_bo.shape),
        ],
        out_specs=pl.BlockSpec((TILE, OUT), lambda i: (i, 0)),
        out_shape=jax.ShapeDtypeStruct((N, OUT), f32),
        compiler_params=pltpu.CompilerParams(
            dimension_semantics=("arbitrary",),
        ),
    )(x, fr, enc_W1, row2(enc_b1), enc_W2, row2(enc_b2),
      pol_W1, row2(pol_b1), pol_W2, row2(pol_b2), pol_W3, row2(pol_b3),
      pol_Wh, row2(pol_bh),
      w0f, w0p, exp_b0, exp_W1, exp_b1, exp_W2, exp_b2, exp_Wo, exp_bo)
    return out
